# baseline (speedup 1.0000x reference)
import functools

import jax
import jax.numpy as jnp
from jax import lax
from jax.experimental import pallas as pl
from jax.experimental.pallas import tpu as pltpu
from jax.experimental.pallas import tpu_sc as plsc

_NUM_SEQS = 16
_HIDDEN = 1024


def _pooler(lens_hbm, hs_hbm, out_hbm, lens_v, idx_v, rows_v, sem):
    c = lax.axis_index("c")
    s = lax.axis_index("s")

    @pl.when(jnp.logical_and(c == 0, s == 0))
    def _():
        pltpu.sync_copy(lens_hbm, lens_v)
        lens = lens_v[...]
        idx_v[...] = plsc.cumsum(lens) - lens
        pltpu.async_copy(hs_hbm.at[idx_v], rows_v, sem).wait()
        pltpu.sync_copy(rows_v, out_hbm)


def kernel(hidden_states, prompt_lens):
    lens_i32 = prompt_lens.astype(jnp.int32)
    mesh = plsc.VectorSubcoreMesh(core_axis_name="c", subcore_axis_name="s")
    run = functools.partial(
        pl.kernel,
        mesh=mesh,
        out_type=jax.ShapeDtypeStruct((_NUM_SEQS, _HIDDEN), jnp.float32),
        scratch_types=[
            pltpu.VMEM((_NUM_SEQS,), jnp.int32),
            pltpu.VMEM((_NUM_SEQS,), jnp.int32),
            pltpu.VMEM((_NUM_SEQS, _HIDDEN), jnp.float32),
            pltpu.SemaphoreType.DMA,
        ],
        compiler_params=pltpu.CompilerParams(needs_layout_passes=False),
    )(_pooler)
    return run(lens_i32, hidden_states)


# speedup vs baseline: 1.0711x; 1.0711x over previous
import functools

import jax
import jax.numpy as jnp
from jax import lax
from jax.experimental import pallas as pl
from jax.experimental.pallas import tpu as pltpu
from jax.experimental.pallas import tpu_sc as plsc

_NUM_SEQS = 16
_HIDDEN = 1024


def _pooler(lens_hbm, hs_hbm, out_hbm, lens_v, idx_v, rows_v, sem):
    c = lax.axis_index("c")
    s = lax.axis_index("s")

    @pl.when(jnp.logical_and(c == 0, s == 0))
    def _():
        pltpu.sync_copy(lens_hbm, lens_v)
        lens = lens_v[...]
        idx_v[...] = plsc.cumsum(lens) - lens
        pltpu.async_copy(hs_hbm.at[idx_v], rows_v, sem).wait()
        pltpu.sync_copy(rows_v, out_hbm)


def kernel(hidden_states, prompt_lens):
    lens_i32 = prompt_lens.astype(jnp.int32)
    mesh = plsc.VectorSubcoreMesh(
        core_axis_name="c", subcore_axis_name="s", num_cores=1
    )
    run = functools.partial(
        pl.kernel,
        mesh=mesh,
        out_type=jax.ShapeDtypeStruct((_NUM_SEQS, _HIDDEN), jnp.float32),
        scratch_types=[
            pltpu.VMEM((_NUM_SEQS,), jnp.int32),
            pltpu.VMEM((_NUM_SEQS,), jnp.int32),
            pltpu.VMEM((_NUM_SEQS, _HIDDEN), jnp.float32),
            pltpu.SemaphoreType.DMA,
        ],
        compiler_params=pltpu.CompilerParams(needs_layout_passes=False),
    )(_pooler)
    return run(lens_i32, hidden_states)


# speedup vs baseline: 1.0833x; 1.0114x over previous
import functools

import jax
import jax.numpy as jnp
from jax import lax
from jax.experimental import pallas as pl
from jax.experimental.pallas import tpu as pltpu
from jax.experimental.pallas import tpu_sc as plsc

_NUM_SEQS = 16
_HIDDEN = 1024


def _pooler(lens_hbm, hs_hbm, out_hbm, lens_v, idx_v, rows_v, sem):
    c = lax.axis_index("c")
    s = lax.axis_index("s")

    @pl.when(jnp.logical_and(c == 0, s == 0))
    def _():
        pltpu.sync_copy(lens_hbm, lens_v)
        lens = lens_v[...]
        idx_v[...] = plsc.cumsum(lens) - lens
        pltpu.async_copy(hs_hbm.at[idx_v], rows_v, sem).wait()
        pltpu.sync_copy(rows_v, out_hbm)


def kernel(hidden_states, prompt_lens):
    lens_i32 = prompt_lens.astype(jnp.int32)
    mesh = plsc.VectorSubcoreMesh(
        core_axis_name="c", subcore_axis_name="s", num_cores=1
    )
    run = functools.partial(
        pl.kernel,
        mesh=mesh,
        out_type=jax.ShapeDtypeStruct((_NUM_SEQS, _HIDDEN), jnp.float32),
        scratch_types=[
            pltpu.VMEM((_NUM_SEQS,), jnp.int32),
            pltpu.VMEM((_NUM_SEQS,), jnp.int32),
            pltpu.VMEM((_NUM_SEQS, _HIDDEN), jnp.float32),
            pltpu.SemaphoreType.DMA,
        ],
        compiler_params=pltpu.CompilerParams(
            needs_layout_passes=False, skip_device_barrier=True
        ),
    )(_pooler)
    return run(lens_i32, hidden_states)


# speedup vs baseline: 1.0844x; 1.0010x over previous
import functools

import jax
import jax.numpy as jnp
from jax import lax
from jax.experimental import pallas as pl
from jax.experimental.pallas import tpu as pltpu
from jax.experimental.pallas import tpu_sc as plsc

_NUM_SEQS = 16
_HIDDEN = 1024


def _pooler(lens_hbm, hs_hbm, out_hbm, lens_v, idx_v, rows_v, sem):
    c = lax.axis_index("c")
    s = lax.axis_index("s")

    @pl.when(jnp.logical_and(c == 0, s == 0))
    def _():
        pltpu.sync_copy(lens_hbm, lens_v)
        lens = lens_v[...]
        idx_v[...] = plsc.cumsum(lens) - lens
        pltpu.async_copy(hs_hbm.at[idx_v], rows_v, sem).wait()
        pltpu.sync_copy(rows_v, out_hbm)


def kernel(hidden_states, prompt_lens):
    lens_i32 = prompt_lens.astype(jnp.int32)
    mesh = plsc.VectorSubcoreMesh(
        core_axis_name="c", subcore_axis_name="s", num_cores=1, num_subcores=1
    )
    run = functools.partial(
        pl.kernel,
        mesh=mesh,
        out_type=jax.ShapeDtypeStruct((_NUM_SEQS, _HIDDEN), jnp.float32),
        scratch_types=[
            pltpu.VMEM((_NUM_SEQS,), jnp.int32),
            pltpu.VMEM((_NUM_SEQS,), jnp.int32),
            pltpu.VMEM((_NUM_SEQS, _HIDDEN), jnp.float32),
            pltpu.SemaphoreType.DMA,
        ],
        compiler_params=pltpu.CompilerParams(
            needs_layout_passes=False, skip_device_barrier=True
        ),
    )(_pooler)
    return run(lens_i32, hidden_states)


# speedup vs baseline: 1.1088x; 1.0225x over previous
import functools

import jax
import jax.numpy as jnp
from jax import lax
from jax.experimental import pallas as pl
from jax.experimental.pallas import tpu as pltpu
from jax.experimental.pallas import tpu_sc as plsc

_NUM_SEQS = 16
_HIDDEN = 1024


def _pooler(lens_hbm, hs_hbm, out_hbm, lens_s, sem):
    pltpu.sync_copy(lens_hbm, lens_s)
    copies = []
    offset = jnp.int32(0)
    for i in range(_NUM_SEQS):
        copy = pltpu.make_async_copy(
            hs_hbm.at[pl.ds(offset, 1)], out_hbm.at[pl.ds(i, 1)], sem
        )
        copy.start()
        copies.append(copy)
        offset = offset + lens_s[i]
    for copy in copies:
        copy.wait()


def kernel(hidden_states, prompt_lens):
    lens_i32 = prompt_lens.astype(jnp.int32)
    mesh = plsc.ScalarSubcoreMesh(axis_name="c", num_cores=1)
    run = functools.partial(
        pl.kernel,
        mesh=mesh,
        out_type=jax.ShapeDtypeStruct((_NUM_SEQS, _HIDDEN), jnp.float32),
        scratch_types=[
            pltpu.SMEM((_NUM_SEQS,), jnp.int32),
            pltpu.SemaphoreType.DMA,
        ],
        compiler_params=pltpu.CompilerParams(needs_layout_passes=False),
    )(_pooler)
    return run(lens_i32, hidden_states)


# speedup vs baseline: 1.1129x; 1.0037x over previous
import functools

import jax
import jax.numpy as jnp
from jax import lax
from jax.experimental import pallas as pl
from jax.experimental.pallas import tpu as pltpu
from jax.experimental.pallas import tpu_sc as plsc

_NUM_SEQS = 16
_HIDDEN = 1024


def _pooler(lens_hbm, hs_hbm, out_hbm, lens_s, sem):
    pltpu.sync_copy(lens_hbm, lens_s)
    copies = []
    offset = jnp.int32(0)
    for i in range(_NUM_SEQS):
        copy = pltpu.make_async_copy(
            hs_hbm.at[pl.ds(offset, 1)], out_hbm.at[pl.ds(i, 1)], sem
        )
        copy.start()
        copies.append(copy)
        offset = offset + lens_s[i]
    for copy in copies:
        copy.wait()


def kernel(hidden_states, prompt_lens):
    lens_i32 = prompt_lens.astype(jnp.int32)
    mesh = plsc.ScalarSubcoreMesh(axis_name="c", num_cores=1)
    run = functools.partial(
        pl.kernel,
        mesh=mesh,
        out_type=jax.ShapeDtypeStruct((_NUM_SEQS, _HIDDEN), jnp.float32),
        scratch_types=[
            pltpu.SMEM((_NUM_SEQS,), jnp.int32),
            pltpu.SemaphoreType.DMA,
        ],
        compiler_params=pltpu.CompilerParams(
            needs_layout_passes=False, skip_device_barrier=True
        ),
    )(_pooler)
    return run(lens_i32, hidden_states)


# speedup vs baseline: 1.1130x; 1.0001x over previous
import functools

import jax
import jax.numpy as jnp
from jax import lax
from jax.experimental import pallas as pl
from jax.experimental.pallas import tpu as pltpu
from jax.experimental.pallas import tpu_sc as plsc

_NUM_SEQS = 16
_HIDDEN = 1024


def _pooler(lens_hbm, hs_hbm, out_hbm, lens_s, sem):
    pltpu.sync_copy(lens_hbm, lens_s)
    copies = []
    offset = jnp.int32(0)
    for i in range(_NUM_SEQS):
        copy = pltpu.make_async_copy(
            hs_hbm.at[pl.ds(offset, 1)], out_hbm.at[pl.ds(i, 1)], sem
        )
        copy.start()
        copies.append(copy)
        offset = offset + lens_s[i]
    for copy in copies:
        copy.wait()


def kernel(hidden_states, prompt_lens):
    lens_i32 = prompt_lens.astype(jnp.int32)
    mesh = plsc.ScalarSubcoreMesh(axis_name="c", num_cores=1)
    run = functools.partial(
        pl.kernel,
        mesh=mesh,
        out_type=jax.ShapeDtypeStruct((_NUM_SEQS, _HIDDEN), jnp.float32),
        scratch_types=[
            pltpu.SMEM((_NUM_SEQS,), jnp.int32),
            pltpu.SemaphoreType.DMA,
        ],
    )(_pooler)
    return run(lens_i32, hidden_states)


# speedup vs baseline: 1.1133x; 1.0003x over previous
import functools

import jax
import jax.numpy as jnp
from jax import lax
from jax.experimental import pallas as pl
from jax.experimental.pallas import tpu as pltpu
from jax.experimental.pallas import tpu_sc as plsc

_NUM_SEQS = 16
_HIDDEN = 1024


def _pooler(lens_hbm, hs_hbm, out_hbm, lens_s, lens_sem, row_sem):
    lens_copy = pltpu.make_async_copy(lens_hbm, lens_s, lens_sem)
    lens_copy.start()
    copies = [
        pltpu.make_async_copy(
            hs_hbm.at[pl.ds(jnp.int32(0), 1)], out_hbm.at[pl.ds(0, 1)], row_sem
        )
    ]
    copies[0].start()
    lens_copy.wait()
    offset = lens_s[0]
    for i in range(1, _NUM_SEQS):
        copy = pltpu.make_async_copy(
            hs_hbm.at[pl.ds(offset, 1)], out_hbm.at[pl.ds(i, 1)], row_sem
        )
        copy.start()
        copies.append(copy)
        offset = offset + lens_s[i]
    for copy in copies:
        copy.wait()


def kernel(hidden_states, prompt_lens):
    lens_i32 = prompt_lens.astype(jnp.int32)
    mesh = plsc.ScalarSubcoreMesh(axis_name="c", num_cores=1)
    run = functools.partial(
        pl.kernel,
        mesh=mesh,
        out_type=jax.ShapeDtypeStruct((_NUM_SEQS, _HIDDEN), jnp.float32),
        scratch_types=[
            pltpu.SMEM((_NUM_SEQS,), jnp.int32),
            pltpu.SemaphoreType.DMA,
            pltpu.SemaphoreType.DMA,
        ],
    )(_pooler)
    return run(lens_i32, hidden_states)


# speedup vs baseline: 1.1166x; 1.0029x over previous
import functools

import jax
import jax.numpy as jnp
from jax.experimental import pallas as pl
from jax.experimental.pallas import tpu as pltpu
from jax.experimental.pallas import tpu_sc as plsc

_NUM_SEQS = 16
_HIDDEN = 1024


def _pooler(lens_hbm, hs_hbm, out_hbm, lens_s, lens_sem, row_sem):
    lens_copy = pltpu.make_async_copy(lens_hbm, lens_s, lens_sem)
    lens_copy.start()
    copies = [
        pltpu.make_async_copy(
            hs_hbm.at[pl.ds(jnp.int32(0), 1)], out_hbm.at[pl.ds(0, 1)], row_sem
        )
    ]
    copies[0].start()
    lens_copy.wait()
    offset = lens_s[0]
    for i in range(1, _NUM_SEQS):
        copy = pltpu.make_async_copy(
            hs_hbm.at[pl.ds(offset, 1)], out_hbm.at[pl.ds(i, 1)], row_sem
        )
        copy.start()
        copies.append(copy)
        offset = offset + lens_s[i]
    for copy in copies:
        copy.wait()


def kernel(hidden_states, prompt_lens):
    lens_i32 = prompt_lens.astype(jnp.int32)
    mesh = plsc.ScalarSubcoreMesh(axis_name="c", num_cores=1)
    run = functools.partial(
        pl.kernel,
        mesh=mesh,
        out_type=jax.ShapeDtypeStruct((_NUM_SEQS, _HIDDEN), jnp.float32),
        scratch_types=[
            pltpu.SMEM((_NUM_SEQS,), jnp.int32),
            pltpu.SemaphoreType.DMA,
            pltpu.SemaphoreType.DMA,
        ],
    )(_pooler)
    return run(lens_i32, hidden_states)
